# baseline (speedup 1.0000x reference)
import functools

import jax
import jax.numpy as jnp
from jax import lax
from jax.experimental import pallas as pl
from jax.experimental.pallas import tpu as pltpu
from jax.experimental.pallas import tpu_sc as plsc

_N_USERS = 50000
_N_ITEMS = 25000
_N_ENT = 50000
_D = 64
_E = 800000

_NC = 2
_NS = 16
_L = 16
_NCH = _D // _L

_K = 128
_E_PAD = 802816
_E_T = _E_PAD // _NS
_BLK_T = _E_T // _K

_N_STAGE = 28
_E_S = _E_T // _N_STAGE
_NBLK_S = _E_S // _K
_NBUF = 7
_NGRP = _NBLK_S // _NBUF
_WB = 32

_NP = 75264


def _make_fused():
    acc_rows = _NP + 128
    z_per_tile = acc_rows // _NS
    w_per_tile = _NP // _NS
    mesh = plsc.VectorSubcoreMesh(core_axis_name="c", subcore_axis_name="s")

    scratch = [
        pltpu.VMEM((2, _E_S), jnp.int32),
        pltpu.VMEM((2, _NBLK_S, _K), jnp.int32),
        pltpu.VMEM((2, _E_S), jnp.float32),
        pltpu.VMEM((_NBUF, _K, _L), jnp.float32),
        pltpu.VMEM((_NBUF, _K, _L), jnp.float32),
        pltpu.VMEM((_WB, _L), jnp.float32),
        pltpu.VMEM((2, _WB, _L), jnp.float32),
        pltpu.VMEM((2, _WB, _L), jnp.float32),
        pltpu.VMEM((2, _WB, _L), jnp.float32),
        pltpu.VMEM((2, _WB), jnp.int32),
        pltpu.VMEM((2, _WB), jnp.int32),
        pltpu.VMEM_SHARED((acc_rows, _L), jnp.float32),
    ] + [pltpu.SemaphoreType.DMA] * (2 * _NBUF + 7)

    @functools.partial(
        pl.kernel,
        mesh=mesh,
        out_type=(jax.ShapeDtypeStruct((_NP * _NCH, _L), jnp.float32),
                  jax.ShapeDtypeStruct((_NP * _NCH, _L), jnp.float32)),
        scratch_types=scratch,
        compiler_params=pltpu.CompilerParams(
            use_tc_tiling_on_sc=False, needs_layout_passes=False),
    )
    def fused(*refs):
        (tab, colb, rowb, wb, zsrc, h1out, fin,
         colbuf, rowbuf, wvbuf, data, sdata, zstage, wbuf, xbuf, hbuf,
         widx_r, widx_w, acc, *sems) = refs
        gsem = sems[:_NBUF]
        ssem = sems[_NBUF:2 * _NBUF]
        psem = sems[2 * _NBUF]
        wsem = sems[2 * _NBUF + 1:2 * _NBUF + 3]
        xsem = sems[2 * _NBUF + 3:2 * _NBUF + 5]
        hsem = sems[2 * _NBUF + 5:2 * _NBUF + 7]
        cid = lax.axis_index("c")
        sid = lax.axis_index("s")
        e0 = sid * _E_T
        eblk0 = sid * _BLK_T
        o0 = sid * w_per_tile
        pltpu.sync_copy(zsrc, zstage)

        def prefetch(stage, slot):
            sbase = e0 + stage * _E_S
            sblk = eblk0 + stage * _NBLK_S
            pltpu.async_copy(colb.at[pl.ds(sbase, _E_S)],
                             colbuf.at[slot], psem)
            pltpu.async_copy(rowb.at[pl.ds(sblk, _NBLK_S)],
                             rowbuf.at[slot], psem)
            pltpu.async_copy(wb.at[pl.ds(sbase, _E_S)],
                             wvbuf.at[slot], psem)

        def prefetch_wait(slot):
            pltpu.make_async_copy(colb.at[pl.ds(0, _E_S)],
                                  colbuf.at[slot], psem).wait()
            pltpu.make_async_copy(rowb.at[pl.ds(0, _NBLK_S)],
                                  rowbuf.at[slot], psem).wait()
            pltpu.make_async_copy(wb.at[pl.ds(0, _E_S)],
                                  wvbuf.at[slot], psem).wait()

        def scatter_phase(src_tab, chunk):
            prefetch(0, 0)

            @pl.loop(0, _N_STAGE)
            def _stage(stage):
                slot = jnp.bitwise_and(stage, 1)
                prefetch_wait(slot)

                @pl.when(stage < _N_STAGE - 1)
                def _():
                    prefetch(stage + 1, 1 - slot)

                gidx_s = colbuf.at[slot]
                row_s = rowbuf.at[slot]
                wv_s = wvbuf.at[slot]

                @pl.loop(0, _E_S // _L, unroll=4)
                def _bi(j):
                    o = j * _L
                    gidx_s[pl.ds(o, _L)] = gidx_s[pl.ds(o, _L)] * _NCH + chunk

                for b in range(_NBUF):
                    pltpu.async_copy(
                        src_tab.at[gidx_s.at[pl.ds(b * _K, _K)]],
                        data.at[b], gsem[b])

                @pl.loop(0, _NGRP)
                def _grp(g):
                    for b in range(_NBUF):
                        blk = g * _NBUF + b
                        pltpu.make_async_copy(
                            src_tab.at[gidx_s.at[pl.ds(blk * _K, _K)]],
                            data.at[b], gsem[b]).wait()
                        db = data.at[b]
                        sb = sdata.at[b]

                        @pl.when(g > 0)
                        def _():
                            pltpu.make_async_copy(
                                sb, acc.at[row_s.at[blk - _NBUF]],
                                ssem[b]).wait()

                        @pl.loop(0, _K // _L)
                        def _wg(j):
                            wv16 = wv_s[pl.ds(blk * _K + j * _L, _L)]
                            for i in range(_L):
                                e = j * _L + i
                                wvec = wv16[jnp.full((_L,), i, jnp.int32)]
                                sb[e] = db[e] * wvec
                        pltpu.async_copy(sb, acc.at[row_s.at[blk]],
                                         ssem[b], add=True)

                        @pl.when(blk + _NBUF < _NBLK_S)
                        def _():
                            pltpu.async_copy(
                                src_tab.at[gidx_s.at[
                                    pl.ds((blk + _NBUF) * _K, _K)]],
                                data.at[b], gsem[b])

                for b in range(_NBUF):
                    lastblk = (_NGRP - 1) * _NBUF + b
                    pltpu.make_async_copy(
                        sdata.at[b], acc.at[row_s.at[lastblk]],
                        ssem[b]).wait()

        def build_widx(bufb, off, chunk):
            iota4 = lax.iota(jnp.int32, _L) * _NCH
            base = off * _NCH + chunk
            for j in range(_WB // _L):
                bufb[pl.ds(j * _L, _L)] = iota4 + (base + j * _L * _NCH)

        def wb_plain(dst, chunk):
            def fill(i, b):
                off = o0 + i * _WB
                wib = widx_w.at[b]
                build_widx(wib, off, chunk)
                pltpu.sync_copy(acc.at[pl.ds(off, _WB)], wbuf.at[b])
                pltpu.sync_copy(zstage, acc.at[pl.ds(off, _WB)])
                pltpu.async_copy(wbuf.at[b], dst.at[wib], wsem[b])

            def wait(b):
                pltpu.make_async_copy(wbuf.at[b], dst.at[widx_w.at[b]],
                                      wsem[b]).wait()

            nblk = w_per_tile // _WB
            for b in range(2):
                fill(b, b)

            @pl.loop(0, (nblk - 2) // 2)
            def _wb(i):
                for b in range(2):
                    wait(b)
                    fill(2 + i * 2 + b, b)
            wait(0)
            fill(nblk - 1, 0)
            for b in range(2):
                wait(b)

        def wb_combine(dst, xr, h1r, chunk, p):
            def readxh(i, b):
                off = o0 + i * _WB
                rib = widx_r.at[b]
                build_widx(rib, off, chunk)
                pltpu.async_copy(xr.at[rib], xbuf.at[b], xsem[b])
                pltpu.async_copy(h1r.at[rib], hbuf.at[b], hsem[b])

            def slot(i, b, do_wait, do_prefetch):
                off = o0 + i * _WB
                pltpu.make_async_copy(xr.at[widx_r.at[b]], xbuf.at[b],
                                      xsem[b]).wait()
                pltpu.make_async_copy(h1r.at[widx_r.at[b]], hbuf.at[b],
                                      hsem[b]).wait()
                if do_wait:
                    pltpu.make_async_copy(wbuf.at[b], dst.at[widx_w.at[b]],
                                          wsem[b]).wait()
                wib = widx_w.at[b]
                build_widx(wib, off, chunk)
                pltpu.sync_copy(acc.at[pl.ds(off, _WB)], wbuf.at[b])

                @pl.when(p == 0)
                def _():
                    pltpu.sync_copy(zstage, acc.at[pl.ds(off, _WB)])

                wbb, xbb, hbb = wbuf.at[b], xbuf.at[b], hbuf.at[b]

                @pl.loop(0, _WB, unroll=4)
                def _cmb(r):
                    wbb[r] = (wbb[r] + xbb[r] + hbb[r]) * (1.0 / 3.0)
                pltpu.async_copy(wbuf.at[b], dst.at[wib], wsem[b])
                if do_prefetch:
                    readxh(i + 2, b)

            nblk = w_per_tile // _WB
            for b in range(2):
                readxh(b, b)
            for b in range(2):
                slot(b, b, False, True)

            @pl.loop(0, (nblk - 4) // 2)
            def _wc(i):
                for b in range(2):
                    slot(2 + i * 2 + b, b, True, True)
            slot(nblk - 3, 0, True, True)
            slot(nblk - 2, 1, True, False)
            slot(nblk - 1, 0, True, False)
            for b in range(2):
                pltpu.make_async_copy(wbuf.at[b], dst.at[widx_w.at[b]],
                                      wsem[b]).wait()

        @pl.loop(0, 2)
        def _pass(p):
            chunk = cid * 2 + p

            @pl.when(p == 0)
            def _():
                r0 = sid * z_per_tile
                nz_full, nz_rem = z_per_tile // _WB, z_per_tile % _WB

                @pl.loop(0, nz_full)
                def _z(i):
                    pltpu.sync_copy(zstage, acc.at[pl.ds(r0 + i * _WB, _WB)])
                if nz_rem:
                    pltpu.sync_copy(zstage.at[pl.ds(0, nz_rem)],
                                    acc.at[pl.ds(r0 + nz_full * _WB, nz_rem)])
            plsc.subcore_barrier()

            scatter_phase(tab, chunk)
            plsc.subcore_barrier()
            wb_plain(h1out, chunk)
            plsc.subcore_barrier()
            scatter_phase(h1out, chunk)
            plsc.subcore_barrier()
            wb_combine(fin, tab, h1out, chunk, p)
            plsc.subcore_barrier()

    return fused


def _pad_edges(row, col, w):
    pad = _E_PAD - _E
    row_p = jnp.concatenate(
        [row, jnp.full((pad,), _NP, jnp.int32)]).reshape(_E_PAD // _K, _K)
    col_p = jnp.concatenate([col, jnp.zeros((pad,), jnp.int32)])
    w_p = jnp.concatenate([w, jnp.zeros((pad,), jnp.float32)])
    return row_p, col_p, w_p


def kernel(entity_emb, user_emb, kg_edge_index, kg_edge_weight,
           ui_edge_index, ui_edge_weight):
    f32 = jnp.float32
    zsrc = jnp.zeros((_WB, _L), f32)
    fused = _make_fused()

    krow_p, kcol_p, kw_p = _pad_edges(
        kg_edge_index[0], kg_edge_index[1], kg_edge_weight)
    xe4 = jnp.concatenate(
        [entity_emb, jnp.zeros((_NP - _N_ENT, _D), f32)]
    ).reshape(_NP * _NCH, _L)
    _, ent_fin = fused(xe4, kcol_p, krow_p, kw_p, zsrc)
    entity_out = ent_fin.reshape(_NP, _D)[:_N_ENT]

    urow_p, ucol_p, uw_p = _pad_edges(
        ui_edge_index[0], ui_edge_index[1], ui_edge_weight)
    ui4 = jnp.concatenate(
        [user_emb.reshape(_N_USERS * _NCH, _L),
         ent_fin[:_N_ITEMS * _NCH],
         jnp.zeros(((_NP - _N_USERS - _N_ITEMS) * _NCH, _L), f32)])
    _, ui_fin = fused(ui4, ucol_p, urow_p, uw_p, zsrc)
    user_out = ui_fin.reshape(_NP, _D)[:_N_USERS]

    return (user_out, entity_out)


# speedup vs baseline: 1.4147x; 1.4147x over previous
import functools

import jax
import jax.numpy as jnp
from jax import lax
from jax.experimental import pallas as pl
from jax.experimental.pallas import tpu as pltpu
from jax.experimental.pallas import tpu_sc as plsc

_N_USERS = 50000
_N_ITEMS = 25000
_N_ENT = 50000
_D = 64
_E = 800000

_NC = 2
_NS = 16
_L = 16
_NCH = _D // _L

_K = 128
_E_PAD = 802816
_E_T = _E_PAD // _NS
_BLK_T = _E_T // _K

_N_STAGE = 14
_E_S = _E_T // _N_STAGE
_NBLK_S = _E_S // _K
_NBUF = 7
_NGRP = _NBLK_S // _NBUF
_WB = 64

_NP = 75264


def _make_spmm():
    acc_rows = _NP + 128
    z_per_tile = acc_rows // _NS
    w_per_tile = _NP // _NS
    mesh = plsc.VectorSubcoreMesh(core_axis_name="c", subcore_axis_name="s")

    scratch = [
        pltpu.VMEM((2, _E_S), jnp.int32),
        pltpu.VMEM((2, _NBLK_S, _K), jnp.int32),
        pltpu.VMEM((2, _E_S), jnp.float32),
        pltpu.VMEM((_NBUF, _K, _L), jnp.float32),
        pltpu.VMEM((_NBUF, _K, _L), jnp.float32),
        pltpu.VMEM((_WB, _L), jnp.float32),
        pltpu.VMEM((2, _WB, _L), jnp.float32),
        pltpu.VMEM_SHARED((acc_rows, _L), jnp.float32),
    ] + [pltpu.SemaphoreType.DMA] * (2 * _NBUF + 3)

    @functools.partial(
        pl.kernel,
        mesh=mesh,
        out_type=jax.ShapeDtypeStruct((_NP, _NCH, _L), jnp.float32),
        scratch_types=scratch,
        compiler_params=pltpu.CompilerParams(
            use_tc_tiling_on_sc=False, needs_layout_passes=False),
    )
    def spmm(*refs):
        (tab, colb, rowb, wb, zsrc, out,
         colbuf, rowbuf, wvbuf, data, sdata, zstage, wbuf, acc, *sems) = refs
        gsem = sems[:_NBUF]
        ssem = sems[_NBUF:2 * _NBUF]
        psem = sems[2 * _NBUF]
        wsem = sems[2 * _NBUF + 1:2 * _NBUF + 3]
        cid = lax.axis_index("c")
        sid = lax.axis_index("s")
        e0 = sid * _E_T
        eblk0 = sid * _BLK_T
        pltpu.sync_copy(zsrc, zstage)

        def prefetch(stage, slot):
            sbase = e0 + stage * _E_S
            sblk = eblk0 + stage * _NBLK_S
            pltpu.async_copy(colb.at[pl.ds(sbase, _E_S)],
                             colbuf.at[slot], psem)
            pltpu.async_copy(rowb.at[pl.ds(sblk, _NBLK_S)],
                             rowbuf.at[slot], psem)
            pltpu.async_copy(wb.at[pl.ds(sbase, _E_S)],
                             wvbuf.at[slot], psem)

        def prefetch_wait(slot):
            pltpu.make_async_copy(colb.at[pl.ds(0, _E_S)],
                                  colbuf.at[slot], psem).wait()
            pltpu.make_async_copy(rowb.at[pl.ds(0, _NBLK_S)],
                                  rowbuf.at[slot], psem).wait()
            pltpu.make_async_copy(wb.at[pl.ds(0, _E_S)],
                                  wvbuf.at[slot], psem).wait()

        for p in range(2):
            chunk = cid * 2 + p

            if p == 0:
                r0 = sid * z_per_tile
                nz_full, nz_rem = z_per_tile // _WB, z_per_tile % _WB

                @pl.loop(0, nz_full)
                def _z(i):
                    pltpu.sync_copy(zstage, acc.at[pl.ds(r0 + i * _WB, _WB)])
                if nz_rem:
                    pltpu.sync_copy(zstage.at[pl.ds(0, nz_rem)],
                                    acc.at[pl.ds(r0 + nz_full * _WB, nz_rem)])
            plsc.subcore_barrier()

            prefetch(0, 0)

            @pl.loop(0, _N_STAGE)
            def _stage(stage):
                slot = jnp.bitwise_and(stage, 1)
                prefetch_wait(slot)

                @pl.when(stage < _N_STAGE - 1)
                def _():
                    prefetch(stage + 1, 1 - slot)

                gidx_s = colbuf.at[slot]
                row_s = rowbuf.at[slot]
                wv_s = wvbuf.at[slot]

                @pl.loop(0, _E_S // _L, unroll=4)
                def _bi(j):
                    o = j * _L
                    gidx_s[pl.ds(o, _L)] = gidx_s[pl.ds(o, _L)] * _NCH + chunk

                for b in range(_NBUF):
                    pltpu.async_copy(
                        tab.at[gidx_s.at[pl.ds(b * _K, _K)]],
                        data.at[b], gsem[b])

                @pl.loop(0, _NGRP)
                def _grp(g):
                    for b in range(_NBUF):
                        blk = g * _NBUF + b
                        pltpu.make_async_copy(
                            tab.at[gidx_s.at[pl.ds(blk * _K, _K)]],
                            data.at[b], gsem[b]).wait()
                        db = data.at[b]
                        sb = sdata.at[b]

                        @pl.when(g > 0)
                        def _():
                            pltpu.make_async_copy(
                                sb, acc.at[row_s.at[blk - _NBUF]],
                                ssem[b]).wait()

                        @pl.loop(0, _K // _L)
                        def _wg(j):
                            wv16 = wv_s[pl.ds(blk * _K + j * _L, _L)]
                            for i in range(_L):
                                e = j * _L + i
                                wvec = wv16[jnp.full((_L,), i, jnp.int32)]
                                sb[e] = db[e] * wvec
                        pltpu.async_copy(sb, acc.at[row_s.at[blk]],
                                         ssem[b], add=True)

                        @pl.when(blk + _NBUF < _NBLK_S)
                        def _():
                            pltpu.async_copy(
                                tab.at[gidx_s.at[
                                    pl.ds((blk + _NBUF) * _K, _K)]],
                                data.at[b], gsem[b])

                for b in range(_NBUF):
                    lastblk = (_NGRP - 1) * _NBUF + b
                    pltpu.make_async_copy(
                        sdata.at[b], acc.at[row_s.at[lastblk]],
                        ssem[b]).wait()

            plsc.subcore_barrier()

            o0 = sid * w_per_tile
            nw_full, nw_rem = w_per_tile // _WB, w_per_tile % _WB

            def wb_fill(i, b):
                off = o0 + i * _WB
                pltpu.sync_copy(acc.at[pl.ds(off, _WB)], wbuf.at[b])
                if p == 0:
                    pltpu.sync_copy(zstage, acc.at[pl.ds(off, _WB)])
                pltpu.async_copy(wbuf.at[b],
                                 out.at[pl.ds(off, _WB), chunk], wsem[b])

            def wb_wait(b):
                pltpu.make_async_copy(wbuf.at[b],
                                      out.at[pl.ds(o0, _WB), chunk],
                                      wsem[b]).wait()

            for b in range(2):
                wb_fill(b, b)

            npairs = (nw_full - 2) // 2

            @pl.loop(0, npairs)
            def _wb(i):
                for b in range(2):
                    wb_wait(b)
                    wb_fill(2 + i * 2 + b, b)

            for k in range(nw_full - 2 - 2 * npairs):
                b = k % 2
                wb_wait(b)
                wb_fill(2 + 2 * npairs + k, b)

            for b in range(2):
                wb_wait(b)
            if nw_rem:
                off = o0 + nw_full * _WB
                pltpu.sync_copy(acc.at[pl.ds(off, nw_rem)],
                                wbuf.at[0, pl.ds(0, nw_rem)])
                if p == 0:
                    pltpu.sync_copy(zstage.at[pl.ds(0, nw_rem)],
                                    acc.at[pl.ds(off, nw_rem)])
                pltpu.sync_copy(wbuf.at[0, pl.ds(0, nw_rem)],
                                out.at[pl.ds(off, nw_rem), chunk])
            plsc.subcore_barrier()

    return spmm


def _combine_body(x_ref, a_ref, b_ref, o_ref):
    o_ref[...] = (x_ref[...] + a_ref[...] + b_ref[...]) * (1.0 / 3.0)


def _combine(x, a, b):
    rows = x.shape[0]
    br = 256
    grid = rows // br
    spec = pl.BlockSpec((br, 128), lambda i: (i, 0))
    return pl.pallas_call(
        _combine_body,
        grid=(grid,),
        in_specs=[spec, spec, spec],
        out_specs=spec,
        out_shape=jax.ShapeDtypeStruct((rows, 128), jnp.float32),
    )(x, a, b)


def _pad_edges(row, col, w):
    pad = _E_PAD - _E
    row_p = jnp.concatenate(
        [row, jnp.full((pad,), _NP, jnp.int32)]).reshape(_E_PAD // _K, _K)
    col_p = jnp.concatenate([col, jnp.zeros((pad,), jnp.int32)])
    w_p = jnp.concatenate([w, jnp.zeros((pad,), jnp.float32)])
    return row_p, col_p, w_p


def kernel(entity_emb, user_emb, kg_edge_index, kg_edge_weight,
           ui_edge_index, ui_edge_weight):
    f32 = jnp.float32
    zsrc = jnp.zeros((_WB, _L), f32)
    spmm = _make_spmm()

    krow_p, kcol_p, kw_p = _pad_edges(
        kg_edge_index[0], kg_edge_index[1], kg_edge_weight)
    x_ent = jnp.concatenate(
        [entity_emb, jnp.zeros((_NP - _N_ENT, _D), f32)])
    s1 = spmm(x_ent.reshape(_NP * _NCH, _L), kcol_p, krow_p, kw_p, zsrc)
    s2 = spmm(s1.reshape(_NP * _NCH, _L), kcol_p, krow_p, kw_p, zsrc)
    ent_full = _combine(x_ent.reshape(-1, 128), s1.reshape(-1, 128),
                        s2.reshape(-1, 128))
    entity_out = ent_full.reshape(_NP, _D)[:_N_ENT]

    urow_p, ucol_p, uw_p = _pad_edges(
        ui_edge_index[0], ui_edge_index[1], ui_edge_weight)
    ui_x = jnp.concatenate(
        [user_emb, entity_out[:_N_ITEMS],
         jnp.zeros((_NP - _N_USERS - _N_ITEMS, _D), f32)])
    u1 = spmm(ui_x.reshape(_NP * _NCH, _L), ucol_p, urow_p, uw_p, zsrc)
    u2 = spmm(u1.reshape(_NP * _NCH, _L), ucol_p, urow_p, uw_p, zsrc)
    ui_full = _combine(ui_x.reshape(-1, 128), u1.reshape(-1, 128),
                       u2.reshape(-1, 128))
    user_out = ui_full.reshape(_NP, _D)[:_N_USERS]

    return (user_out, entity_out)
